# baseline (speedup 1.0000x reference)
import functools

import jax
import jax.numpy as jnp
from jax import lax
from jax.experimental import pallas as pl
from jax.experimental.pallas import tpu as pltpu
from jax.experimental.pallas import tpu_sc as plsc

_MINI = 1e-06


def _sc_gather(exp_m, idx):
    D, K = exp_m.shape
    B = idx.shape[0]
    b_per_w = 16
    NW_USED = B // b_per_w
    mesh = plsc.VectorSubcoreMesh(core_axis_name="c", subcore_axis_name="s")

    @functools.partial(
        pl.kernel,
        out_type=jax.ShapeDtypeStruct((B, K), jnp.float32),
        mesh=mesh,
        compiler_params=pltpu.CompilerParams(needs_layout_passes=False),
        scratch_types=[
            pltpu.VMEM((b_per_w,), jnp.int32),
            pltpu.VMEM((b_per_w, K), jnp.float32),
        ],
    )
    def gather_k(exp_m_hbm, idx_hbm, out_hbm, idx_v, rows_v):
        wid = lax.axis_index("s") * 2 + lax.axis_index("c")
        lane_iota = lax.broadcasted_iota(jnp.int32, (16,), 0)

        @pl.when(wid < NW_USED)
        def _():
            base = wid * b_per_w
            pltpu.sync_copy(idx_hbm.at[pl.ds(base, b_per_w)], idx_v)
            iv = idx_v[...]

            def gbody(j, carry):
                row = jnp.sum(jnp.where(lane_iota == j, iv, 0))
                pltpu.sync_copy(exp_m_hbm.at[pl.ds(row, 1)],
                                rows_v.at[pl.ds(j, 1)])
                return carry

            lax.fori_loop(0, b_per_w, gbody, 0)
            pltpu.sync_copy(rows_v, out_hbm.at[pl.ds(base, b_per_w)])

    return gather_k(exp_m, idx)


def _tc_dense(g_m, exp_n, beta, alpha2, bow, idx_r, idx_c, scal):
    B, K = g_m.shape
    V = exp_n.shape[0]
    f32 = jnp.float32
    hi = lax.Precision.HIGHEST

    def body(gm_ref, en_ref, be_ref, al_ref, bow_ref, ir_ref, ic_ref, sc_ref,
             rows_ref, eno_ref, qz_ref):
        rho = sc_ref[0, 0]
        omr = sc_ref[0, 1]
        rho_scaled = sc_ref[0, 2]

        gm = gm_ref[...]
        en = en_ref[...]
        be = be_ref[...]
        c = be + en
        s = jnp.sum(c, axis=0, keepdims=True)
        w = (al_ref[...] + gm) / s
        bow = bow_ref[...].astype(f32)
        mask = (bow > 0.0).astype(f32)

        Z = lax.dot_general(w, c, (((1,), (1,)), ((), ())),
                            precision=hi, preferred_element_type=f32)
        Zm = Z + _MINI
        inv = 1.0 / Zm
        P = mask * inv
        Rw = bow * inv

        RC = lax.dot_general(Rw, c, (((1,), (0,)), ((), ())),
                             precision=hi, preferred_element_type=f32)
        temp_m = w * RC
        RW = lax.dot_general(Rw, w, (((0,), (0,)), ((), ())),
                             precision=hi, preferred_element_type=f32)
        eno_ref[...] = omr * en + rho_scaled * (c * RW)

        PC = lax.dot_general(P, c, (((1,), (0,)), ((), ())),
                             precision=hi, preferred_element_type=f32)
        PW = lax.dot_general(P, w, (((0,), (0,)), ((), ())),
                             precision=hi, preferred_element_type=f32)
        t1 = jnp.sum(w * jnp.log(w) * PC)
        t2 = jnp.sum(c * jnp.log(c) * PW)
        t3 = -jnp.sum(mask * (Z * inv) * jnp.log(Zm))
        qz_ref[0, 0] = t1 + t2 + t3

        new_rows = omr * gm + rho * temp_m
        eq = ir_ref[...] == ic_ref[...]
        bi = lax.broadcasted_iota(jnp.int32, (B, B), 1)
        last = jnp.max(jnp.where(eq, bi, -1), axis=1, keepdims=True)
        onehot = (bi == last).astype(f32)
        rows_ref[...] = lax.dot_general(onehot, new_rows,
                                        (((1,), (0,)), ((), ())),
                                        precision=hi, preferred_element_type=f32)

    vm = pl.BlockSpec(memory_space=pltpu.VMEM)
    return pl.pallas_call(
        body,
        in_specs=[vm, vm, vm, vm, vm, vm, vm,
                  pl.BlockSpec(memory_space=pltpu.SMEM)],
        out_specs=[vm, vm, pl.BlockSpec(memory_space=pltpu.SMEM)],
        out_shape=[
            jax.ShapeDtypeStruct((B, K), f32),
            jax.ShapeDtypeStruct((V, K), f32),
            jax.ShapeDtypeStruct((1, 1), f32),
        ],
    )(g_m, exp_n, beta, alpha2, bow, idx_r, idx_c, scal)


def _sc_copy_scatter(exp_m, idx, rows):
    D, K = exp_m.shape
    B = idx.shape[0]
    NW = 32
    rows_per_w = -(-D // NW)
    rows_per_w += (-rows_per_w) % 8
    mesh = plsc.VectorSubcoreMesh(core_axis_name="c", subcore_axis_name="s")

    @functools.partial(
        pl.kernel,
        out_type=jax.ShapeDtypeStruct((D, K), jnp.float32),
        mesh=mesh,
        compiler_params=pltpu.CompilerParams(needs_layout_passes=False),
        scratch_types=[
            pltpu.VMEM((B,), jnp.int32),
            pltpu.VMEM((B, K), jnp.float32),
        ],
    )
    def scatter_k(exp_m_hbm, idx_hbm, rows_hbm, out_hbm, idx_v, rows_v):
        wid = lax.axis_index("s") * 2 + lax.axis_index("c")
        lo = wid * rows_per_w
        pltpu.sync_copy(idx_hbm, idx_v)
        pltpu.sync_copy(rows_hbm, rows_v)

        n_full = D // rows_per_w
        rem = D - n_full * rows_per_w

        @pl.when(wid < n_full)
        def _():
            pltpu.sync_copy(exp_m_hbm.at[pl.ds(lo, rows_per_w)],
                            out_hbm.at[pl.ds(lo, rows_per_w)])

        if rem:
            @pl.when(wid == n_full)
            def _():
                pltpu.sync_copy(exp_m_hbm.at[pl.ds(n_full * rows_per_w, rem)],
                                out_hbm.at[pl.ds(n_full * rows_per_w, rem)])
        lane_iota = lax.broadcasted_iota(jnp.int32, (16,), 0)

        def sbody(b, carry):
            grp = (b // 16) * 16
            lane = b - grp
            iv = idx_v[pl.ds(grp, 16)]
            idx_b = jnp.sum(jnp.where(lane_iota == lane, iv, 0))

            @pl.when(jnp.logical_and(idx_b >= lo, idx_b < lo + rows_per_w))
            def _():
                pltpu.sync_copy(rows_v.at[pl.ds(b, 1)],
                                out_hbm.at[pl.ds(idx_b, 1)])

            return carry

        lax.fori_loop(0, B, sbody, 0)

    return scatter_k(exp_m, idx, rows)


def kernel(exp_m, exp_n, alpha, beta, batch_BOW, batch_indices, iter_n):
    D, K = exp_m.shape
    B = batch_BOW.shape[0]
    rho64 = 1.0 / jnp.asarray(iter_n + 5, dtype=jnp.float64) ** 0.9
    rho = rho64.astype(jnp.float32)
    omr = (1.0 - rho64).astype(jnp.float32)
    scal = jnp.reshape(jnp.stack([rho, omr, rho * (float(D) / float(B))]), (1, 3))

    g_m = _sc_gather(exp_m, batch_indices)
    rows, exp_n_new, qz = _tc_dense(
        g_m, exp_n, beta,
        jnp.reshape(alpha, (1, K)),
        batch_BOW,
        jnp.reshape(batch_indices, (B, 1)),
        jnp.reshape(batch_indices, (1, B)),
        scal,
    )
    exp_m_new = _sc_copy_scatter(exp_m, batch_indices, rows)
    return exp_m_new, exp_n_new, qz[0, 0]


# speedup vs baseline: 9.7411x; 9.7411x over previous
import functools

import jax
import jax.numpy as jnp
from jax import lax
from jax.experimental import pallas as pl
from jax.experimental.pallas import tpu as pltpu
from jax.experimental.pallas import tpu_sc as plsc

_MINI = 1e-06


def _sc_gather(exp_m, idx):
    D, K = exp_m.shape
    B = idx.shape[0]
    b_per_w = 16
    NW_USED = B // b_per_w
    mesh = plsc.VectorSubcoreMesh(core_axis_name="c", subcore_axis_name="s")

    @functools.partial(
        pl.kernel,
        out_type=jax.ShapeDtypeStruct((B, K), jnp.float32),
        mesh=mesh,
        compiler_params=pltpu.CompilerParams(needs_layout_passes=False),
        scratch_types=[
            pltpu.VMEM((b_per_w,), jnp.int32),
            pltpu.VMEM((b_per_w, K), jnp.float32),
        ],
    )
    def gather_k(exp_m_hbm, idx_hbm, out_hbm, idx_v, rows_v):
        wid = lax.axis_index("s") * 2 + lax.axis_index("c")
        lane_iota = lax.broadcasted_iota(jnp.int32, (16,), 0)

        @pl.when(wid < NW_USED)
        def _():
            base = wid * b_per_w
            pltpu.sync_copy(idx_hbm.at[pl.ds(base, b_per_w)], idx_v)
            iv = idx_v[...]

            def gbody(j, carry):
                row = jnp.sum(jnp.where(lane_iota == j, iv, 0))
                pltpu.sync_copy(exp_m_hbm.at[pl.ds(row, 1)],
                                rows_v.at[pl.ds(j, 1)])
                return carry

            lax.fori_loop(0, b_per_w, gbody, 0)
            pltpu.sync_copy(rows_v, out_hbm.at[pl.ds(base, b_per_w)])

    return gather_k(exp_m, idx)


def _tc_dense(g_m, exp_n, beta, alpha2, bow, idx_r, idx_c, scal):
    B, K = g_m.shape
    V = exp_n.shape[0]
    f32 = jnp.float32
    hi = lax.Precision.HIGHEST

    def body(gm_ref, en_ref, be_ref, al_ref, bow_ref, ir_ref, ic_ref, sc_ref,
             rows_ref, eno_ref, qz_ref):
        rho = sc_ref[0, 0]
        omr = sc_ref[0, 1]
        rho_scaled = sc_ref[0, 2]

        gm = gm_ref[...]
        en = en_ref[...]
        be = be_ref[...]
        c = be + en
        s = jnp.sum(c, axis=0, keepdims=True)
        w = (al_ref[...] + gm) / s
        bow = bow_ref[...].astype(f32)
        mask = (bow > 0.0).astype(f32)

        Z = lax.dot_general(w, c, (((1,), (1,)), ((), ())),
                            precision=hi, preferred_element_type=f32)
        Zm = Z + _MINI
        inv = 1.0 / Zm
        P = mask * inv
        Rw = bow * inv

        RC = lax.dot_general(Rw, c, (((1,), (0,)), ((), ())),
                             precision=hi, preferred_element_type=f32)
        temp_m = w * RC
        RW = lax.dot_general(Rw, w, (((0,), (0,)), ((), ())),
                             precision=hi, preferred_element_type=f32)
        eno_ref[...] = omr * en + rho_scaled * (c * RW)

        PC = lax.dot_general(P, c, (((1,), (0,)), ((), ())),
                             precision=hi, preferred_element_type=f32)
        PW = lax.dot_general(P, w, (((0,), (0,)), ((), ())),
                             precision=hi, preferred_element_type=f32)
        t1 = jnp.sum(w * jnp.log(w) * PC)
        t2 = jnp.sum(c * jnp.log(c) * PW)
        t3 = -jnp.sum(mask * (Z * inv) * jnp.log(Zm))
        qz_ref[0, 0] = t1 + t2 + t3

        new_rows = omr * gm + rho * temp_m
        eq = ir_ref[...] == ic_ref[...]
        bi = lax.broadcasted_iota(jnp.int32, (B, B), 1)
        last = jnp.max(jnp.where(eq, bi, -1), axis=1, keepdims=True)
        onehot = (bi == last).astype(f32)
        rows_ref[...] = lax.dot_general(onehot, new_rows,
                                        (((1,), (0,)), ((), ())),
                                        precision=hi, preferred_element_type=f32)

    vm = pl.BlockSpec(memory_space=pltpu.VMEM)
    return pl.pallas_call(
        body,
        in_specs=[vm, vm, vm, vm, vm, vm, vm,
                  pl.BlockSpec(memory_space=pltpu.SMEM)],
        out_specs=[vm, vm, pl.BlockSpec(memory_space=pltpu.SMEM)],
        out_shape=[
            jax.ShapeDtypeStruct((B, K), f32),
            jax.ShapeDtypeStruct((V, K), f32),
            jax.ShapeDtypeStruct((1, 1), f32),
        ],
    )(g_m, exp_n, beta, alpha2, bow, idx_r, idx_c, scal)


def _sc_copy_scatter(exp_m, idx, rows):
    D, K = exp_m.shape
    B = idx.shape[0]
    NW = 32
    CHUNK = 512
    n_full = D // CHUNK
    rem = D - n_full * CHUNK
    n_chunks = n_full + (1 if rem else 0)
    nj = -(-n_chunks // NW)
    mesh = plsc.VectorSubcoreMesh(core_axis_name="c", subcore_axis_name="s")

    @functools.partial(
        pl.kernel,
        out_type=jax.ShapeDtypeStruct((D, K), jnp.float32),
        mesh=mesh,
        compiler_params=pltpu.CompilerParams(needs_layout_passes=False),
        scratch_types=[
            pltpu.VMEM((B,), jnp.int32),
            pltpu.VMEM((B, K), jnp.float32),
            pltpu.VMEM((CHUNK, K), jnp.float32),
        ],
    )
    def scatter_k(exp_m_hbm, idx_hbm, rows_hbm, out_hbm, idx_v, rows_v, buf):
        wid = lax.axis_index("s") * 2 + lax.axis_index("c")
        pltpu.sync_copy(idx_hbm, idx_v)
        pltpu.sync_copy(rows_hbm, rows_v)

        for j in range(nj):
            c = wid + NW * j

            @pl.when(c < n_full)
            def _():
                off = c * CHUNK
                pltpu.sync_copy(exp_m_hbm.at[pl.ds(off, CHUNK)], buf)
                pltpu.sync_copy(buf, out_hbm.at[pl.ds(off, CHUNK)])

            if rem:
                @pl.when(c == n_full)
                def _():
                    off = n_full * CHUNK
                    pltpu.sync_copy(exp_m_hbm.at[pl.ds(off, rem)],
                                    buf.at[pl.ds(0, rem)])
                    pltpu.sync_copy(buf.at[pl.ds(0, rem)],
                                    out_hbm.at[pl.ds(off, rem)])

        lane_iota = lax.broadcasted_iota(jnp.int32, (16,), 0)

        def sbody(b, carry):
            grp = (b // 16) * 16
            lane = b - grp
            iv = idx_v[pl.ds(grp, 16)]
            idx_b = jnp.sum(jnp.where(lane_iota == lane, iv, 0))
            c_of_b = idx_b // CHUNK
            owner = c_of_b - (c_of_b // NW) * NW

            @pl.when(owner == wid)
            def _():
                pltpu.sync_copy(rows_v.at[pl.ds(b, 1)],
                                out_hbm.at[pl.ds(idx_b, 1)])

            return carry

        lax.fori_loop(0, B, sbody, 0)

    return scatter_k(exp_m, idx, rows)


def kernel(exp_m, exp_n, alpha, beta, batch_BOW, batch_indices, iter_n):
    D, K = exp_m.shape
    B = batch_BOW.shape[0]
    rho64 = 1.0 / jnp.asarray(iter_n + 5, dtype=jnp.float64) ** 0.9
    rho = rho64.astype(jnp.float32)
    omr = (1.0 - rho64).astype(jnp.float32)
    scal = jnp.reshape(jnp.stack([rho, omr, rho * (float(D) / float(B))]), (1, 3))

    g_m = _sc_gather(exp_m, batch_indices)
    rows, exp_n_new, qz = _tc_dense(
        g_m, exp_n, beta,
        jnp.reshape(alpha, (1, K)),
        batch_BOW,
        jnp.reshape(batch_indices, (B, 1)),
        jnp.reshape(batch_indices, (1, B)),
        scal,
    )
    exp_m_new = _sc_copy_scatter(exp_m, batch_indices, rows)
    return exp_m_new, exp_n_new, qz[0, 0]


# speedup vs baseline: 11.0398x; 1.1333x over previous
import functools

import jax
import jax.numpy as jnp
from jax import lax
from jax.experimental import pallas as pl
from jax.experimental.pallas import tpu as pltpu
from jax.experimental.pallas import tpu_sc as plsc

_MINI = 1e-06


def _sc_gather(exp_m, idx):
    D, K = exp_m.shape
    B = idx.shape[0]
    b_per_w = 16
    NW_USED = B // b_per_w
    mesh = plsc.VectorSubcoreMesh(core_axis_name="c", subcore_axis_name="s")

    @functools.partial(
        pl.kernel,
        out_type=jax.ShapeDtypeStruct((B, K), jnp.float32),
        mesh=mesh,
        compiler_params=pltpu.CompilerParams(needs_layout_passes=False),
        scratch_types=[
            pltpu.VMEM((b_per_w,), jnp.int32),
            pltpu.VMEM((b_per_w, K), jnp.float32),
        ],
    )
    def gather_k(exp_m_hbm, idx_hbm, out_hbm, idx_v, rows_v):
        wid = lax.axis_index("s") * 2 + lax.axis_index("c")
        lane_iota = lax.broadcasted_iota(jnp.int32, (16,), 0)

        @pl.when(wid < NW_USED)
        def _():
            base = wid * b_per_w
            pltpu.sync_copy(idx_hbm.at[pl.ds(base, b_per_w)], idx_v)
            iv = idx_v[...]

            def gbody(j, carry):
                row = jnp.sum(jnp.where(lane_iota == j, iv, 0))
                pltpu.sync_copy(exp_m_hbm.at[pl.ds(row, 1)],
                                rows_v.at[pl.ds(j, 1)])
                return carry

            lax.fori_loop(0, b_per_w, gbody, 0)
            pltpu.sync_copy(rows_v, out_hbm.at[pl.ds(base, b_per_w)])

    return gather_k(exp_m, idx)


def _tc_dense(g_m, exp_n, beta, alpha2, bow, idx_r, idx_c, scal):
    B, K = g_m.shape
    V = exp_n.shape[0]
    f32 = jnp.float32
    hi = lax.Precision.HIGHEST

    def body(gm_ref, en_ref, be_ref, al_ref, bow_ref, ir_ref, ic_ref, sc_ref,
             rows_ref, eno_ref, qz_ref):
        rho = sc_ref[0, 0]
        omr = sc_ref[0, 1]
        rho_scaled = sc_ref[0, 2]

        gm = gm_ref[...]
        en = en_ref[...]
        be = be_ref[...]
        c = be + en
        s = jnp.sum(c, axis=0, keepdims=True)
        w = (al_ref[...] + gm) / s
        bow = bow_ref[...].astype(f32)
        mask = (bow > 0.0).astype(f32)

        Z = lax.dot_general(w, c, (((1,), (1,)), ((), ())),
                            precision=hi, preferred_element_type=f32)
        Zm = Z + _MINI
        inv = 1.0 / Zm
        P = mask * inv
        Rw = bow * inv

        RC = lax.dot_general(Rw, c, (((1,), (0,)), ((), ())),
                             precision=hi, preferred_element_type=f32)
        temp_m = w * RC
        RW = lax.dot_general(Rw, w, (((0,), (0,)), ((), ())),
                             precision=hi, preferred_element_type=f32)
        eno_ref[...] = omr * en + rho_scaled * (c * RW)

        PC = lax.dot_general(P, c, (((1,), (0,)), ((), ())),
                             precision=hi, preferred_element_type=f32)
        PW = lax.dot_general(P, w, (((0,), (0,)), ((), ())),
                             precision=hi, preferred_element_type=f32)
        t1 = jnp.sum(w * jnp.log(w) * PC)
        t2 = jnp.sum(c * jnp.log(c) * PW)
        t3 = -jnp.sum(mask * (Z * inv) * jnp.log(Zm))
        qz_ref[0, 0] = t1 + t2 + t3

        new_rows = omr * gm + rho * temp_m
        eq = ir_ref[...] == ic_ref[...]
        bi = lax.broadcasted_iota(jnp.int32, (B, B), 1)
        last = jnp.max(jnp.where(eq, bi, -1), axis=1, keepdims=True)
        onehot = (bi == last).astype(f32)
        rows_ref[...] = lax.dot_general(onehot, new_rows,
                                        (((1,), (0,)), ((), ())),
                                        precision=hi, preferred_element_type=f32)

    vm = pl.BlockSpec(memory_space=pltpu.VMEM)
    return pl.pallas_call(
        body,
        in_specs=[vm, vm, vm, vm, vm, vm, vm,
                  pl.BlockSpec(memory_space=pltpu.SMEM)],
        out_specs=[vm, vm, pl.BlockSpec(memory_space=pltpu.SMEM)],
        out_shape=[
            jax.ShapeDtypeStruct((B, K), f32),
            jax.ShapeDtypeStruct((V, K), f32),
            jax.ShapeDtypeStruct((1, 1), f32),
        ],
    )(g_m, exp_n, beta, alpha2, bow, idx_r, idx_c, scal)


def _sc_copy(exp_m):
    D, K = exp_m.shape
    NW = 32
    CHUNK = 512
    n_full = D // CHUNK
    rem = D - n_full * CHUNK
    n_chunks = n_full + (1 if rem else 0)
    nj = -(-n_chunks // NW)
    mesh = plsc.VectorSubcoreMesh(core_axis_name="c", subcore_axis_name="s")

    @functools.partial(
        pl.kernel,
        out_type=jax.ShapeDtypeStruct((D, K), jnp.float32),
        mesh=mesh,
        compiler_params=pltpu.CompilerParams(needs_layout_passes=False),
        scratch_types=[
            pltpu.VMEM((CHUNK, K), jnp.float32),
        ],
    )
    def copy_k(exp_m_hbm, out_hbm, buf):
        wid = lax.axis_index("s") * 2 + lax.axis_index("c")
        for j in range(nj):
            c = wid + NW * j

            @pl.when(c < n_full)
            def _():
                off = c * CHUNK
                pltpu.sync_copy(exp_m_hbm.at[pl.ds(off, CHUNK)], buf)
                pltpu.sync_copy(buf, out_hbm.at[pl.ds(off, CHUNK)])

            if rem:
                @pl.when(c == n_full)
                def _():
                    off = n_full * CHUNK
                    pltpu.sync_copy(exp_m_hbm.at[pl.ds(off, rem)],
                                    buf.at[pl.ds(0, rem)])
                    pltpu.sync_copy(buf.at[pl.ds(0, rem)],
                                    out_hbm.at[pl.ds(off, rem)])

    return copy_k(exp_m)


def _sc_scatter(idx, rows, out_ref):
    B, K = rows.shape
    NW = 32
    b_per_w = B // NW
    mesh = plsc.VectorSubcoreMesh(core_axis_name="c", subcore_axis_name="s")

    @functools.partial(
        pl.kernel,
        out_type=(),
        mesh=mesh,
        compiler_params=pltpu.CompilerParams(needs_layout_passes=False),
        scratch_types=[
            pltpu.VMEM((B,), jnp.int32),
            pltpu.VMEM((B, K), jnp.float32),
        ],
    )
    def scatter_k(idx_hbm, rows_hbm, out_hbm, idx_v, rows_v):
        wid = lax.axis_index("s") * 2 + lax.axis_index("c")
        pltpu.sync_copy(idx_hbm, idx_v)
        pltpu.sync_copy(rows_hbm, rows_v)
        lane_iota = lax.broadcasted_iota(jnp.int32, (16,), 0)

        def sbody(b, carry):
            grp = (b // 16) * 16
            lane = b - grp
            iv = idx_v[pl.ds(grp, 16)]
            idx_b = jnp.sum(jnp.where(lane_iota == lane, iv, 0))
            pltpu.sync_copy(rows_v.at[pl.ds(b, 1)],
                            out_hbm.at[pl.ds(idx_b, 1)])
            return carry

        lax.fori_loop(wid * b_per_w, (wid + 1) * b_per_w, sbody, 0)

    scatter_k(idx, rows, out_ref)


def kernel(exp_m, exp_n, alpha, beta, batch_BOW, batch_indices, iter_n):
    D, K = exp_m.shape
    B = batch_BOW.shape[0]
    rho64 = 1.0 / jnp.asarray(iter_n + 5, dtype=jnp.float64) ** 0.9
    rho = rho64.astype(jnp.float32)
    omr = (1.0 - rho64).astype(jnp.float32)
    scal = jnp.reshape(jnp.stack([rho, omr, rho * (float(D) / float(B))]), (1, 3))

    g_m = _sc_gather(exp_m, batch_indices)
    copied = _sc_copy(exp_m)
    rows, exp_n_new, qz = _tc_dense(
        g_m, exp_n, beta,
        jnp.reshape(alpha, (1, K)),
        batch_BOW,
        jnp.reshape(batch_indices, (B, 1)),
        jnp.reshape(batch_indices, (1, B)),
        scal,
    )
    buf = jax.new_ref(copied)
    _sc_scatter(batch_indices, rows, buf)
    return buf[...], exp_n_new, qz[0, 0]


# speedup vs baseline: 11.4255x; 1.0349x over previous
import functools

import jax
import jax.numpy as jnp
from jax import lax
from jax.experimental import pallas as pl
from jax.experimental.pallas import tpu as pltpu
from jax.experimental.pallas import tpu_sc as plsc

_MINI = 1e-06


def _sc_gather(exp_m, idx):
    D, K = exp_m.shape
    B = idx.shape[0]
    b_per_w = 16
    NW_USED = B // b_per_w
    mesh = plsc.VectorSubcoreMesh(core_axis_name="c", subcore_axis_name="s")

    @functools.partial(
        pl.kernel,
        out_type=jax.ShapeDtypeStruct((B, K), jnp.float32),
        mesh=mesh,
        compiler_params=pltpu.CompilerParams(needs_layout_passes=False),
        scratch_types=[
            pltpu.VMEM((b_per_w,), jnp.int32),
            pltpu.VMEM((b_per_w, K), jnp.float32),
        ],
    )
    def gather_k(exp_m_hbm, idx_hbm, out_hbm, idx_v, rows_v):
        wid = lax.axis_index("s") * 2 + lax.axis_index("c")
        lane_iota = lax.broadcasted_iota(jnp.int32, (16,), 0)

        @pl.when(wid < NW_USED)
        def _():
            base = wid * b_per_w
            pltpu.sync_copy(idx_hbm.at[pl.ds(base, b_per_w)], idx_v)
            iv = idx_v[...]

            def gbody(j, carry):
                row = jnp.sum(jnp.where(lane_iota == j, iv, 0))
                pltpu.sync_copy(exp_m_hbm.at[pl.ds(row, 1)],
                                rows_v.at[pl.ds(j, 1)])
                return carry

            lax.fori_loop(0, b_per_w, gbody, 0)
            pltpu.sync_copy(rows_v, out_hbm.at[pl.ds(base, b_per_w)])

    return gather_k(exp_m, idx)


def _tc_dense(g_m, exp_n, beta, alpha2, bow, idx_r, idx_c, scal):
    B, K = g_m.shape
    V = exp_n.shape[0]
    f32 = jnp.float32
    hi = lax.Precision.HIGHEST

    def body(gm_ref, en_ref, be_ref, al_ref, bow_ref, ir_ref, ic_ref, sc_ref,
             rows_ref, eno_ref, qz_ref):
        rho = sc_ref[0, 0]
        omr = sc_ref[0, 1]
        rho_scaled = sc_ref[0, 2]

        gm = gm_ref[...]
        en = en_ref[...]
        be = be_ref[...]
        c = be + en
        s = jnp.sum(c, axis=0, keepdims=True)
        w = (al_ref[...] + gm) / s
        bow = bow_ref[...].astype(f32)
        mask = (bow > 0.0).astype(f32)

        Z = lax.dot_general(w, c, (((1,), (1,)), ((), ())),
                            precision=hi, preferred_element_type=f32)
        Zm = Z + _MINI
        inv = 1.0 / Zm
        P = mask * inv
        Rw = bow * inv

        RC = lax.dot_general(Rw, c, (((1,), (0,)), ((), ())),
                             precision=hi, preferred_element_type=f32)
        temp_m = w * RC
        RW = lax.dot_general(Rw, w, (((0,), (0,)), ((), ())),
                             precision=hi, preferred_element_type=f32)
        eno_ref[...] = omr * en + rho_scaled * (c * RW)

        PC = lax.dot_general(P, c, (((1,), (0,)), ((), ())),
                             precision=hi, preferred_element_type=f32)
        PW = lax.dot_general(P, w, (((0,), (0,)), ((), ())),
                             precision=hi, preferred_element_type=f32)
        t1 = jnp.sum(w * jnp.log(w) * PC)
        t2 = jnp.sum(c * jnp.log(c) * PW)
        t3 = -jnp.sum(mask * (Z * inv) * jnp.log(Zm))
        qz_ref[0, 0] = t1 + t2 + t3

        new_rows = omr * gm + rho * temp_m
        eq = ir_ref[...] == ic_ref[...]
        bi = lax.broadcasted_iota(jnp.int32, (B, B), 1)
        last = jnp.max(jnp.where(eq, bi, -1), axis=1, keepdims=True)
        onehot = (bi == last).astype(f32)
        rows_ref[...] = lax.dot_general(onehot, new_rows,
                                        (((1,), (0,)), ((), ())),
                                        precision=hi, preferred_element_type=f32)

    vm = pl.BlockSpec(memory_space=pltpu.VMEM)
    return pl.pallas_call(
        body,
        in_specs=[vm, vm, vm, vm, vm, vm, vm,
                  pl.BlockSpec(memory_space=pltpu.SMEM)],
        out_specs=[vm, vm, pl.BlockSpec(memory_space=pltpu.SMEM)],
        out_shape=[
            jax.ShapeDtypeStruct((B, K), f32),
            jax.ShapeDtypeStruct((V, K), f32),
            jax.ShapeDtypeStruct((1, 1), f32),
        ],
    )(g_m, exp_n, beta, alpha2, bow, idx_r, idx_c, scal)


def _sc_copy(exp_m):
    D, K = exp_m.shape
    NW = 32
    CHUNK = 512
    n_full = D // CHUNK
    rem = D - n_full * CHUNK
    n_chunks = n_full + (1 if rem else 0)
    nj = -(-n_chunks // NW)
    mesh = plsc.VectorSubcoreMesh(core_axis_name="c", subcore_axis_name="s")

    @functools.partial(
        pl.kernel,
        out_type=jax.ShapeDtypeStruct((D, K), jnp.float32),
        mesh=mesh,
        compiler_params=pltpu.CompilerParams(needs_layout_passes=False),
        scratch_types=[
            pltpu.VMEM((CHUNK, K), jnp.float32),
        ],
    )
    def copy_k(exp_m_hbm, out_hbm, buf):
        wid = lax.axis_index("s") * 2 + lax.axis_index("c")
        for j in range(nj):
            c = wid + NW * j

            @pl.when(c < n_full)
            def _():
                off = c * CHUNK
                pltpu.sync_copy(exp_m_hbm.at[pl.ds(off, CHUNK)], buf)
                pltpu.sync_copy(buf, out_hbm.at[pl.ds(off, CHUNK)])

            if rem:
                @pl.when(c == n_full)
                def _():
                    off = n_full * CHUNK
                    pltpu.sync_copy(exp_m_hbm.at[pl.ds(off, rem)],
                                    buf.at[pl.ds(0, rem)])
                    pltpu.sync_copy(buf.at[pl.ds(0, rem)],
                                    out_hbm.at[pl.ds(off, rem)])

    return copy_k(exp_m)


def _tc_scatter(copied, idx2, rows):
    D, K = copied.shape
    B = rows.shape[0]

    def body(cop_ref, idx_ref, rows_ref, out_ref, sem):
        del cop_ref

        def sbody(b, carry):
            r = idx_ref[0, b]
            pltpu.make_async_copy(rows_ref.at[pl.ds(b, 1)],
                                  out_ref.at[pl.ds(r, 1)], sem).start()
            return carry

        lax.fori_loop(0, B, sbody, 0)

        def wbody(b, carry):
            r = idx_ref[0, b]
            pltpu.make_async_copy(rows_ref.at[pl.ds(b, 1)],
                                  out_ref.at[pl.ds(r, 1)], sem).wait()
            return carry

        lax.fori_loop(0, B, wbody, 0)

    return pl.pallas_call(
        body,
        in_specs=[pl.BlockSpec(memory_space=pl.ANY),
                  pl.BlockSpec(memory_space=pltpu.SMEM),
                  pl.BlockSpec(memory_space=pltpu.VMEM)],
        out_specs=pl.BlockSpec(memory_space=pl.ANY),
        out_shape=jax.ShapeDtypeStruct((D, K), jnp.float32),
        input_output_aliases={0: 0},
        scratch_shapes=[pltpu.SemaphoreType.DMA],
    )(copied, idx2, rows)


def kernel(exp_m, exp_n, alpha, beta, batch_BOW, batch_indices, iter_n):
    D, K = exp_m.shape
    B = batch_BOW.shape[0]
    rho64 = 1.0 / jnp.asarray(iter_n + 5, dtype=jnp.float64) ** 0.9
    rho = rho64.astype(jnp.float32)
    omr = (1.0 - rho64).astype(jnp.float32)
    scal = jnp.reshape(jnp.stack([rho, omr, rho * (float(D) / float(B))]), (1, 3))

    g_m = _sc_gather(exp_m, batch_indices)
    copied = _sc_copy(exp_m)
    rows, exp_n_new, qz = _tc_dense(
        g_m, exp_n, beta,
        jnp.reshape(alpha, (1, K)),
        batch_BOW,
        jnp.reshape(batch_indices, (B, 1)),
        jnp.reshape(batch_indices, (1, B)),
        scal,
    )
    exp_m_new = _tc_scatter(copied, jnp.reshape(batch_indices, (1, B)), rows)
    return exp_m_new, exp_n_new, qz[0, 0]


# speedup vs baseline: 21.6049x; 1.8909x over previous
import functools

import jax
import jax.numpy as jnp
from jax import lax
from jax.experimental import pallas as pl
from jax.experimental.pallas import tpu as pltpu
from jax.experimental.pallas import tpu_sc as plsc

_MINI = 1e-06
_LANES = 128


def _extract16(vec16, lane):
    lane_iota = lax.broadcasted_iota(jnp.int32, (16,), 0)
    return jnp.sum(jnp.where(lane_iota == lane, vec16, 0))


def _sc_gather_t(exp_m_t, idx):
    K, D = exp_m_t.shape
    B = idx.shape[0]
    NW = 32
    b_per_w = B // NW
    gbases = [g * 16 for g in range(K // 16)] + ([K - 16] if K % 16 else [])
    mesh = plsc.VectorSubcoreMesh(core_axis_name="c", subcore_axis_name="s")

    @functools.partial(
        pl.kernel,
        out_type=jax.ShapeDtypeStruct((B, K), jnp.float32),
        mesh=mesh,
        compiler_params=pltpu.CompilerParams(needs_layout_passes=False),
        scratch_types=[
            pltpu.VMEM((B,), jnp.int32),
            pltpu.VMEM((K, _LANES), jnp.float32),
            pltpu.VMEM((1, K), jnp.float32),
        ],
    )
    def gather_k(exp_m_hbm, idx_hbm, out_hbm, idx_v, tile_v, row_v):
        wid = lax.axis_index("s") * 2 + lax.axis_index("c")
        pltpu.sync_copy(idx_hbm, idx_v)
        lane_iota = lax.broadcasted_iota(jnp.int32, (16,), 0)

        for j in range(b_per_w):
            b = wid * b_per_w + j
            grp = (b // 16) * 16
            iv = idx_v[pl.ds(grp, 16)]
            d = _extract16(iv, b - grp)
            toff = (d // _LANES) * _LANES
            dl = d - toff
            pltpu.sync_copy(exp_m_hbm.at[:, pl.ds(toff, _LANES)], tile_v)
            dl16 = jnp.zeros((16,), jnp.int32) + dl
            for base in gbases:
                vals = plsc.load_gather(tile_v, [lane_iota + base, dl16])
                row_v[0, pl.ds(base, 16)] = vals
            pltpu.sync_copy(row_v, out_hbm.at[pl.ds(b, 1)])

    return gather_k(exp_m_t, idx)


def _tc_dense(gm, en_t, be_t, alpha2, bow, idx_r, idx_c, scal):
    B, K = gm.shape
    V = en_t.shape[1]
    f32 = jnp.float32
    hi = lax.Precision.HIGHEST

    def body(gm_ref, en_ref, be_ref, al_ref, bow_ref, ir_ref, ic_ref, sc_ref,
             rows_ref, eno_ref, qz_ref):
        rho = sc_ref[0, 0]
        omr = sc_ref[0, 1]
        rho_scaled = sc_ref[0, 2]

        gm = gm_ref[...]
        c = be_ref[...] + en_ref[...]
        s = jnp.sum(c, axis=1)[None, :]
        w = (al_ref[...] + gm) / s
        bow = bow_ref[...].astype(f32)
        mask = (bow > 0.0).astype(f32)

        Z = lax.dot_general(w, c, (((1,), (0,)), ((), ())),
                            precision=hi, preferred_element_type=f32)
        Zm = Z + _MINI
        inv = 1.0 / Zm
        P = mask * inv
        Rw = bow * inv

        RC = lax.dot_general(Rw, c, (((1,), (1,)), ((), ())),
                             precision=hi, preferred_element_type=f32)
        temp_m = w * RC
        RWt = lax.dot_general(w, Rw, (((0,), (0,)), ((), ())),
                              precision=hi, preferred_element_type=f32)
        eno_ref[...] = omr * en_ref[...] + rho_scaled * (c * RWt)

        PC = lax.dot_general(P, c, (((1,), (1,)), ((), ())),
                             precision=hi, preferred_element_type=f32)
        PWt = lax.dot_general(w, P, (((0,), (0,)), ((), ())),
                              precision=hi, preferred_element_type=f32)
        t1 = jnp.sum(w * jnp.log(w) * PC)
        t2 = jnp.sum(c * jnp.log(c) * PWt)
        t3 = -jnp.sum(mask * (Z * inv) * jnp.log(Zm))
        qz_ref[0, 0] = t1 + t2 + t3

        new_rows = omr * gm + rho * temp_m
        eq = ir_ref[...] == ic_ref[...]
        bi = lax.broadcasted_iota(jnp.int32, (B, B), 1)
        last = jnp.max(jnp.where(eq, bi, -1), axis=1, keepdims=True)
        onehot = (bi == last).astype(f32)
        rows_ref[...] = lax.dot_general(onehot, new_rows,
                                        (((1,), (0,)), ((), ())),
                                        precision=hi, preferred_element_type=f32)

    vm = pl.BlockSpec(memory_space=pltpu.VMEM)
    return pl.pallas_call(
        body,
        in_specs=[vm, vm, vm, vm, vm, vm, vm,
                  pl.BlockSpec(memory_space=pltpu.SMEM)],
        out_specs=[vm, vm, pl.BlockSpec(memory_space=pltpu.SMEM)],
        out_shape=[
            jax.ShapeDtypeStruct((B, K), f32),
            jax.ShapeDtypeStruct((K, V), f32),
            jax.ShapeDtypeStruct((1, 1), f32),
        ],
    )(gm, en_t, be_t, alpha2, bow, idx_r, idx_c, scal)


def _sc_copy_scatter_t(exp_m_t, idx, rows):
    K, D = exp_m_t.shape
    B = idx.shape[0]
    NW = 32
    CHUNK = 1536
    aligned_d = (D // _LANES) * _LANES
    n_full = aligned_d // CHUNK
    rem = aligned_d - n_full * CHUNK
    n_chunks = n_full + (1 if rem else 0)
    nj = -(-n_chunks // NW)
    gbases = [g * 16 for g in range(K // 16)] + ([K - 16] if K % 16 else [])
    mesh = plsc.VectorSubcoreMesh(core_axis_name="c", subcore_axis_name="s")

    @functools.partial(
        pl.kernel,
        out_type=jax.ShapeDtypeStruct((K, D), jnp.float32),
        mesh=mesh,
        compiler_params=pltpu.CompilerParams(needs_layout_passes=False),
        scratch_types=[
            pltpu.VMEM((B,), jnp.int32),
            pltpu.VMEM((B, K), jnp.float32),
            pltpu.VMEM((K, CHUNK), jnp.float32),
        ],
    )
    def copy_k(exp_m_hbm, idx_hbm, rows_hbm, out_hbm, idx_v, rows_v, buf):
        wid = lax.axis_index("s") * 2 + lax.axis_index("c")
        pltpu.sync_copy(idx_hbm, idx_v)
        pltpu.sync_copy(rows_hbm, rows_v)
        lane_iota = lax.broadcasted_iota(jnp.int32, (16,), 0)

        def fixup(c):
            for g in range(B // 16):
                iv = idx_v[pl.ds(g * 16, 16)]
                rel = iv - c * CHUNK
                hit = jnp.logical_and(rel >= 0, rel < CHUNK)
                any_hit = jnp.max(jnp.where(hit, 1, 0))

                @pl.when(any_hit > 0)
                def _():
                    def lbody(l, carry):
                        dl = _extract16(rel, l)

                        @pl.when(jnp.logical_and(dl >= 0, dl < CHUNK))
                        def _():
                            b = g * 16 + l
                            dl16 = jnp.zeros((16,), jnp.int32) + dl
                            for base in gbases:
                                vals = rows_v[b, pl.ds(base, 16)]
                                plsc.store_scatter(
                                    buf, [lane_iota + base, dl16], vals)

                        return carry

                    lax.fori_loop(0, 16, lbody, 0)

        for j in range(nj):
            c = wid + NW * j

            @pl.when(c < n_full)
            def _():
                off = c * CHUNK
                pltpu.sync_copy(exp_m_hbm.at[:, pl.ds(off, CHUNK)], buf)
                fixup(c)
                pltpu.sync_copy(buf, out_hbm.at[:, pl.ds(off, CHUNK)])

            if rem:
                @pl.when(c == n_full)
                def _():
                    off = n_full * CHUNK
                    pltpu.sync_copy(exp_m_hbm.at[:, pl.ds(off, rem)],
                                    buf.at[:, pl.ds(0, rem)])
                    fixup(c)
                    pltpu.sync_copy(buf.at[:, pl.ds(0, rem)],
                                    out_hbm.at[:, pl.ds(off, rem)])

    return copy_k(exp_m_t, idx, rows)


def _tc_tail(copied_t, exp_m_t, idx2, rows):
    K, D = copied_t.shape
    B = rows.shape[0]
    blk = D // _LANES
    off = blk * _LANES
    f32 = jnp.float32
    hi = lax.Precision.HIGHEST

    def body(cop_ref, tail_ref, idx_ref, rows_ref, out_ref):
        del cop_ref
        idxc = idx_ref[...]
        idxr = jnp.reshape(idxc, (B, 1))
        bi = lax.broadcasted_iota(jnp.int32, (B, B), 1)
        last = jnp.max(jnp.where(idxr == idxc, bi, -1), axis=1,
                       keepdims=True)
        is_last = (last == lax.broadcasted_iota(jnp.int32, (B, 1), 0))
        lane = off + lax.broadcasted_iota(jnp.int32, (1, _LANES), 1)
        sel = jnp.logical_and(idxr == lane, is_last).astype(f32)
        hit = jnp.max(sel, axis=0, keepdims=True)
        scattered = lax.dot_general(rows_ref[...], sel,
                                    (((0,), (0,)), ((), ())),
                                    precision=hi,
                                    preferred_element_type=f32)
        out_ref[...] = jnp.where(hit > 0, scattered, tail_ref[...])

    vm = pl.BlockSpec(memory_space=pltpu.VMEM)
    return pl.pallas_call(
        body,
        grid=(1,),
        in_specs=[pl.BlockSpec(memory_space=pl.ANY),
                  pl.BlockSpec((K, _LANES), lambda i: (0, blk)),
                  vm, vm],
        out_specs=pl.BlockSpec((K, _LANES), lambda i: (0, blk)),
        out_shape=jax.ShapeDtypeStruct((K, D), jnp.float32),
        input_output_aliases={0: 0},
    )(copied_t, exp_m_t, idx2, rows)


def kernel(exp_m, exp_n, alpha, beta, batch_BOW, batch_indices, iter_n):
    D, K = exp_m.shape
    B = batch_BOW.shape[0]
    rho64 = 1.0 / jnp.asarray(iter_n + 5, dtype=jnp.float64) ** 0.9
    rho = rho64.astype(jnp.float32)
    omr = (1.0 - rho64).astype(jnp.float32)
    scal = jnp.reshape(jnp.stack([rho, omr, rho * (float(D) / float(B))]), (1, 3))

    exp_m_t = exp_m.T
    g_m = _sc_gather_t(exp_m_t, batch_indices)
    rows, exp_n_new_t, qz = _tc_dense(
        g_m, exp_n.T, beta.T,
        jnp.reshape(alpha, (1, K)),
        batch_BOW,
        jnp.reshape(batch_indices, (B, 1)),
        jnp.reshape(batch_indices, (1, B)),
        scal,
    )
    out_t = _sc_copy_scatter_t(exp_m_t, batch_indices, rows)
    if D % _LANES:
        out_t = _tc_tail(out_t, exp_m_t,
                         jnp.reshape(batch_indices, (1, B)), rows)
    return out_t.T, exp_n_new_t.T, qz[0, 0]


# speedup vs baseline: 24.9664x; 1.1556x over previous
import functools

import jax
import jax.numpy as jnp
from jax import lax
from jax.experimental import pallas as pl
from jax.experimental.pallas import tpu as pltpu
from jax.experimental.pallas import tpu_sc as plsc

_MINI = 1e-06
_LANES = 128


def _extract16(vec16, lane):
    lane_iota = lax.broadcasted_iota(jnp.int32, (16,), 0)
    return jnp.sum(jnp.where(lane_iota == lane, vec16, 0))


def _sc_gather_t(exp_m_t, idx):
    K, D = exp_m_t.shape
    B = idx.shape[0]
    NW = 32
    b_per_w = B // NW
    gbases = [g * 16 for g in range(K // 16)] + ([K - 16] if K % 16 else [])
    mesh = plsc.VectorSubcoreMesh(core_axis_name="c", subcore_axis_name="s")

    @functools.partial(
        pl.kernel,
        out_type=jax.ShapeDtypeStruct((B, K), jnp.float32),
        mesh=mesh,
        compiler_params=pltpu.CompilerParams(needs_layout_passes=False),
        scratch_types=[
            pltpu.VMEM((B,), jnp.int32),
            pltpu.VMEM((K, _LANES), jnp.float32),
            pltpu.VMEM((1, K), jnp.float32),
        ],
    )
    def gather_k(exp_m_hbm, idx_hbm, out_hbm, idx_v, tile_v, row_v):
        wid = lax.axis_index("s") * 2 + lax.axis_index("c")
        pltpu.sync_copy(idx_hbm, idx_v)
        lane_iota = lax.broadcasted_iota(jnp.int32, (16,), 0)

        for j in range(b_per_w):
            b = wid * b_per_w + j
            grp = (b // 16) * 16
            iv = idx_v[pl.ds(grp, 16)]
            d = _extract16(iv, b - grp)
            toff = (d // _LANES) * _LANES
            dl = d - toff
            pltpu.sync_copy(exp_m_hbm.at[:, pl.ds(toff, _LANES)], tile_v)
            dl16 = jnp.zeros((16,), jnp.int32) + dl
            for base in gbases:
                vals = plsc.load_gather(tile_v, [lane_iota + base, dl16])
                row_v[0, pl.ds(base, 16)] = vals
            pltpu.sync_copy(row_v, out_hbm.at[pl.ds(b, 1)])

    return gather_k(exp_m_t, idx)


def _tc_dense(gm, en_t, be_t, alpha2, bow, idx_r, idx_c, scal):
    B, K = gm.shape
    V = en_t.shape[1]
    f32 = jnp.float32
    hi = lax.Precision.HIGHEST

    def body(gm_ref, en_ref, be_ref, al_ref, bow_ref, ir_ref, ic_ref, sc_ref,
             rows_ref, eno_ref, qz_ref):
        rho = sc_ref[0, 0]
        omr = sc_ref[0, 1]
        rho_scaled = sc_ref[0, 2]

        gm = gm_ref[...]
        c = be_ref[...] + en_ref[...]
        s = jnp.sum(c, axis=1)[None, :]
        w = (al_ref[...] + gm) / s
        bow = bow_ref[...].astype(f32)
        mask = (bow > 0.0).astype(f32)

        Z = lax.dot_general(w, c, (((1,), (0,)), ((), ())),
                            precision=hi, preferred_element_type=f32)
        Zm = Z + _MINI
        inv = 1.0 / Zm
        P = mask * inv
        Rw = bow * inv

        RC = lax.dot_general(Rw, c, (((1,), (1,)), ((), ())),
                             precision=hi, preferred_element_type=f32)
        temp_m = w * RC
        RWt = lax.dot_general(w, Rw, (((0,), (0,)), ((), ())),
                              precision=hi, preferred_element_type=f32)
        eno_ref[...] = omr * en_ref[...] + rho_scaled * (c * RWt)

        PC = lax.dot_general(P, c, (((1,), (1,)), ((), ())),
                             precision=hi, preferred_element_type=f32)
        PWt = lax.dot_general(w, P, (((0,), (0,)), ((), ())),
                              precision=hi, preferred_element_type=f32)
        t1 = jnp.sum(w * jnp.log(w) * PC)
        t2 = jnp.sum(c * jnp.log(c) * PWt)
        t3 = -jnp.sum(mask * (Z * inv) * jnp.log(Zm))
        qz_ref[0, 0] = t1 + t2 + t3

        new_rows = omr * gm + rho * temp_m
        eq = ir_ref[...] == ic_ref[...]
        bi = lax.broadcasted_iota(jnp.int32, (B, B), 1)
        last = jnp.max(jnp.where(eq, bi, -1), axis=1, keepdims=True)
        onehot = (bi == last).astype(f32)
        rows_ref[...] = lax.dot_general(onehot, new_rows,
                                        (((1,), (0,)), ((), ())),
                                        precision=hi, preferred_element_type=f32)

    vm = pl.BlockSpec(memory_space=pltpu.VMEM)
    return pl.pallas_call(
        body,
        in_specs=[vm, vm, vm, vm, vm, vm, vm,
                  pl.BlockSpec(memory_space=pltpu.SMEM)],
        out_specs=[vm, vm, pl.BlockSpec(memory_space=pltpu.SMEM)],
        out_shape=[
            jax.ShapeDtypeStruct((B, K), f32),
            jax.ShapeDtypeStruct((K, V), f32),
            jax.ShapeDtypeStruct((1, 1), f32),
        ],
    )(gm, en_t, be_t, alpha2, bow, idx_r, idx_c, scal)


def _sc_copy_scatter_t(exp_m_t, idx, rows):
    K, D = exp_m_t.shape
    B = idx.shape[0]
    NW = 32
    NJ = 2
    aligned_d = (D // _LANES) * _LANES
    tiles = aligned_d // _LANES
    CH = (tiles // (NW * NJ)) * _LANES
    NBIG = tiles - (NW * NJ) * (CH // _LANES)
    CHB = CH + _LANES
    big_span = NBIG * CHB
    gbases = [g * 16 for g in range(K // 16)] + ([K - 16] if K % 16 else [])
    mesh = plsc.VectorSubcoreMesh(core_axis_name="c", subcore_axis_name="s")

    @functools.partial(
        pl.kernel,
        out_type=jax.ShapeDtypeStruct((K, D), jnp.float32),
        mesh=mesh,
        compiler_params=pltpu.CompilerParams(needs_layout_passes=False),
        scratch_types=[
            pltpu.VMEM((B,), jnp.int32),
            pltpu.VMEM((B, K), jnp.float32),
            pltpu.VMEM((K, CHB), jnp.float32),
        ],
    )
    def copy_k(exp_m_hbm, idx_hbm, rows_hbm, out_hbm, idx_v, rows_v, buf):
        wid = lax.axis_index("s") * 2 + lax.axis_index("c")
        pltpu.sync_copy(idx_hbm, idx_v)
        pltpu.sync_copy(rows_hbm, rows_v)
        lane_iota = lax.broadcasted_iota(jnp.int32, (16,), 0)

        def fixup(off, size):
            for g in range(B // 16):
                iv = idx_v[pl.ds(g * 16, 16)]
                rel = iv - off
                hit = jnp.logical_and(rel >= 0, rel < size)
                any_hit = jnp.max(jnp.where(hit, 1, 0))

                @pl.when(any_hit > 0)
                def _():
                    def lbody(l, carry):
                        dl = _extract16(rel, l)

                        @pl.when(jnp.logical_and(dl >= 0, dl < size))
                        def _():
                            b = g * 16 + l
                            dl16 = jnp.zeros((16,), jnp.int32) + dl
                            for base in gbases:
                                vals = rows_v[b, pl.ds(base, 16)]
                                plsc.store_scatter(
                                    buf, [lane_iota + base, dl16], vals)

                        return carry

                    lax.fori_loop(0, 16, lbody, 0)

        def turn(off, size):
            pltpu.sync_copy(exp_m_hbm.at[:, pl.ds(off, size)],
                            buf.at[:, pl.ds(0, size)])
            fixup(off, size)
            pltpu.sync_copy(buf.at[:, pl.ds(0, size)],
                            out_hbm.at[:, pl.ds(off, size)])

        for j in range(NJ):
            c = wid + NW * j

            @pl.when(c < NBIG)
            def _():
                turn(c * CHB, CHB)

            @pl.when(c >= NBIG)
            def _():
                turn(big_span + (c - NBIG) * CH, CH)

    return copy_k(exp_m_t, idx, rows)


def _tc_tail(copied_t, exp_m_t, idx2, rows):
    K, D = copied_t.shape
    B = rows.shape[0]
    blk = D // _LANES
    off = blk * _LANES
    f32 = jnp.float32
    hi = lax.Precision.HIGHEST

    def body(cop_ref, tail_ref, idx_ref, rows_ref, out_ref):
        del cop_ref
        idxc = idx_ref[...]
        idxr = jnp.reshape(idxc, (B, 1))
        bi = lax.broadcasted_iota(jnp.int32, (B, B), 1)
        last = jnp.max(jnp.where(idxr == idxc, bi, -1), axis=1,
                       keepdims=True)
        is_last = (last == lax.broadcasted_iota(jnp.int32, (B, 1), 0))
        lane = off + lax.broadcasted_iota(jnp.int32, (1, _LANES), 1)
        sel = jnp.logical_and(idxr == lane, is_last).astype(f32)
        hit = jnp.max(sel, axis=0, keepdims=True)
        scattered = lax.dot_general(rows_ref[...], sel,
                                    (((0,), (0,)), ((), ())),
                                    precision=hi,
                                    preferred_element_type=f32)
        out_ref[...] = jnp.where(hit > 0, scattered, tail_ref[...])

    vm = pl.BlockSpec(memory_space=pltpu.VMEM)
    return pl.pallas_call(
        body,
        grid=(1,),
        in_specs=[pl.BlockSpec(memory_space=pl.ANY),
                  pl.BlockSpec((K, _LANES), lambda i: (0, blk)),
                  vm, vm],
        out_specs=pl.BlockSpec((K, _LANES), lambda i: (0, blk)),
        out_shape=jax.ShapeDtypeStruct((K, D), jnp.float32),
        input_output_aliases={0: 0},
    )(copied_t, exp_m_t, idx2, rows)


def kernel(exp_m, exp_n, alpha, beta, batch_BOW, batch_indices, iter_n):
    D, K = exp_m.shape
    B = batch_BOW.shape[0]
    rho64 = 1.0 / jnp.asarray(iter_n + 5, dtype=jnp.float64) ** 0.9
    rho = rho64.astype(jnp.float32)
    omr = (1.0 - rho64).astype(jnp.float32)
    scal = jnp.reshape(jnp.stack([rho, omr, rho * (float(D) / float(B))]), (1, 3))

    exp_m_t = exp_m.T
    g_m = _sc_gather_t(exp_m_t, batch_indices)
    rows, exp_n_new_t, qz = _tc_dense(
        g_m, exp_n.T, beta.T,
        jnp.reshape(alpha, (1, K)),
        batch_BOW,
        jnp.reshape(batch_indices, (B, 1)),
        jnp.reshape(batch_indices, (1, B)),
        scal,
    )
    out_t = _sc_copy_scatter_t(exp_m_t, batch_indices, rows)
    if D % _LANES:
        out_t = _tc_tail(out_t, exp_m_t,
                         jnp.reshape(batch_indices, (1, B)), rows)
    return out_t.T, exp_n_new_t.T, qz[0, 0]
